# baseline (speedup 1.0000x reference)
import functools

import jax
import jax.numpy as jnp
from jax import lax
from jax.experimental import pallas as pl
from jax.experimental.pallas import tpu as pltpu
from jax.experimental.pallas import tpu_sc as plsc

B, SRC, D = 16, 512, 256
FILT = 256
NBINS = 256
MAXLEN = 2048

_NC = 2
_NS = 16
_NW = _NC * _NS
_ROWS = B * MAXLEN
_RPW = _ROWS // _NW
_CH = 128
_NCHUNK = _RPW // _CH


def _conv_ln(h, w_ref, b_ref, g_ref, be_ref):
    t, d = h.shape
    f = w_ref.shape[1]
    a0 = jnp.dot(h, w_ref[0:d, :], preferred_element_type=jnp.float32,
                 precision=lax.Precision.DEFAULT)
    a1 = jnp.dot(h, w_ref[d:2 * d, :], preferred_element_type=jnp.float32,
                 precision=lax.Precision.DEFAULT)
    a2 = jnp.dot(h, w_ref[2 * d:3 * d, :], preferred_element_type=jnp.float32,
                 precision=lax.Precision.DEFAULT)
    zrow = jnp.zeros((1, f), jnp.float32)
    y = (a1 + jnp.concatenate([zrow, a0[:-1, :]], axis=0)
         + jnp.concatenate([a2[1:, :], zrow], axis=0) + b_ref[...])
    y = jnp.maximum(y, 0.0)
    m = jnp.mean(y, axis=1, keepdims=True)
    v = jnp.mean((y - m) ** 2, axis=1, keepdims=True)
    return (y - m) * lax.rsqrt(v + 1e-5) * g_ref[...] + be_ref[...]


_AB = 4
_BB = 4


def _stage_a_body(x_ref, w1_ref, b1_ref, g1_ref, be1_ref,
                  w2_ref, b2_ref, g2_ref, be2_ref, lw_ref, lb_ref,
                  dp_ref):
    for i in range(_AB):
        h = x_ref[i]
        h = _conv_ln(h, w1_ref, b1_ref, g1_ref, be1_ref)
        h = _conv_ln(h, w2_ref, b2_ref, g2_ref, be2_ref)
        dp = jnp.sum(h * lw_ref[...], axis=1, keepdims=True) + lb_ref[0, 0]
        dp_ref[i] = jnp.transpose(jnp.maximum(dp, 1e-8), (1, 0))


def _idx_body(dur_ref, gidx_ref, mask_ref):
    f32 = jnp.float32
    jj_col = lax.broadcasted_iota(jnp.int32, (SRC, 1), 0)
    ii_row = lax.broadcasted_iota(jnp.int32, (1, SRC), 1)
    umat = (jj_col <= ii_row).astype(f32)
    pos_row = lax.broadcasted_iota(jnp.int32, (1, MAXLEN), 1)
    ones_row = jnp.ones((1, SRC), f32)
    spread = jnp.bitwise_and(pos_row, SRC - 1)
    for i in range(_AB):
        b = pl.program_id(0) * _AB + i
        d_row = dur_ref[i].astype(f32)
        cum_row = jnp.dot(d_row, umat, preferred_element_type=f32,
                          precision=lax.Precision.DEFAULT)
        cum_col = jnp.transpose(cum_row, (1, 0))

        hmat = (cum_col <= pos_row.astype(f32)).astype(f32)
        idx_row = jnp.dot(ones_row, hmat, preferred_element_type=f32,
                          precision=lax.Precision.DEFAULT)
        idx_i = jnp.minimum(idx_row, SRC - 1).astype(jnp.int32)

        invalid = hmat[SRC - 1:SRC, :]
        gidx_ref[i] = jnp.where(invalid == 0.0, idx_i, spread) + b * SRC
        mask_ref[i] = 1.0 - invalid


def _stage_b_body(exp_ref, mask_ref, e_ref, bounds_lo_ref, bounds_hi_ref,
                  w1_ref, b1_ref, g1_ref, be1_ref,
                  w2_ref, b2_ref, g2_ref, be2_ref, lw_ref, lb_ref,
                  emb_t_ref, out_ref, en_ref):
    for i in range(_BB):
        m_col = jnp.transpose(mask_ref[i], (1, 0))
        e = exp_ref[i] * m_col
        h = _conv_ln(e, w1_ref, b1_ref, g1_ref, be1_ref)
        h = _conv_ln(h, w2_ref, b2_ref, g2_ref, be2_ref)
        en = jnp.sum(h * lw_ref[...], axis=1, keepdims=True) + lb_ref[0, 0]
        en_ref[i] = jnp.transpose(en, (1, 0))

        et_row = e_ref[i]
        lo = bounds_lo_ref[...] < et_row
        hi = bounds_hi_ref[...] < et_row
        onehot_t = jnp.where(lo & jnp.logical_not(hi), 1.0, 0.0)
        emb_rows = lax.dot_general(
            onehot_t, emb_t_ref[...], (((0,), (0,)), ((), ())),
            preferred_element_type=jnp.float32)
        out_ref[i] = e + emb_rows


def _sc_gather_body(table_hbm, gidx_hbm, out_hbm, idx_v,
                    rows0, rows1, rows2, gs0, gs1, gs2, ws0, ws1, ws2):
    wid = lax.axis_index("s") * _NC + lax.axis_index("c")
    base = wid * _RPW
    pltpu.sync_copy(gidx_hbm.at[pl.ds(base, _RPW)], idx_v)
    bufs = (rows0, rows1, rows2)
    gsems = (gs0, gs1, gs2)
    wsems = (ws0, ws1, ws2)
    gh, wh = {}, {}
    for c in range(_NCHUNK):
        i = c % 3
        if c >= 3:
            wh[c - 3].wait()
        gh[c] = pltpu.async_copy(
            table_hbm.at[idx_v.at[pl.ds(c * _CH, _CH)]], bufs[i], gsems[i])
        if c >= 1:
            j = (c - 1) % 3
            gh[c - 1].wait()
            wh[c - 1] = pltpu.async_copy(
                bufs[j], out_hbm.at[pl.ds(base + (c - 1) * _CH, _CH)], wsems[j])
    last = _NCHUNK - 1
    gh[last].wait()
    wh[last] = pltpu.async_copy(
        bufs[last % 3], out_hbm.at[pl.ds(base + last * _CH, _CH)], wsems[last % 3])
    for c in (last - 2, last - 1, last):
        wh[c].wait()


def _full_spec(shape):
    return pl.BlockSpec(shape, lambda b: (0,) * len(shape))


def kernel(x, max_len, duration_target, energy_target,
           dp_w1, dp_b1, dp_g1, dp_be1, dp_w2, dp_b2, dp_g2, dp_be2,
           dp_lw, dp_lb, ep_w1, ep_b1, ep_g1, ep_be1, ep_w2, ep_b2,
           ep_g2, ep_be2, ep_lw, ep_lb, emb_table, bounds):
    f32 = jnp.float32

    dur3 = duration_target.reshape(B, 1, SRC)
    e3 = energy_target.reshape(B, 1, MAXLEN)
    bounds_lo = jnp.concatenate(
        [jnp.full((1,), -jnp.inf, f32), bounds]).reshape(NBINS, 1)
    bounds_hi = jnp.concatenate(
        [bounds, jnp.full((1,), jnp.inf, f32)]).reshape(NBINS, 1)

    dp_w1r = dp_w1.reshape(3 * D, FILT)
    dp_w2r = dp_w2.reshape(3 * FILT, FILT)
    ep_w1r = ep_w1.reshape(3 * D, FILT)
    ep_w2r = ep_w2.reshape(3 * FILT, FILT)
    row = lambda a: a.reshape(1, -1)

    gidx3, mask3 = pl.pallas_call(
        _idx_body,
        grid=(B // _AB,),
        in_specs=[pl.BlockSpec((_AB, 1, SRC), lambda b: (b, 0, 0))],
        out_specs=[
            pl.BlockSpec((_AB, 1, MAXLEN), lambda b: (b, 0, 0)),
            pl.BlockSpec((_AB, 1, MAXLEN), lambda b: (b, 0, 0)),
        ],
        out_shape=[
            jax.ShapeDtypeStruct((B, 1, MAXLEN), jnp.int32),
            jax.ShapeDtypeStruct((B, 1, MAXLEN), f32),
        ],
    )(dur3)

    dp3 = pl.pallas_call(
        _stage_a_body,
        grid=(B // _AB,),
        in_specs=[
            pl.BlockSpec((_AB, SRC, D), lambda b: (b, 0, 0)),
            _full_spec((3 * D, FILT)), _full_spec((1, FILT)),
            _full_spec((1, FILT)), _full_spec((1, FILT)),
            _full_spec((3 * FILT, FILT)), _full_spec((1, FILT)),
            _full_spec((1, FILT)), _full_spec((1, FILT)),
            _full_spec((1, FILT)), _full_spec((1, 1)),
        ],
        out_specs=pl.BlockSpec((_AB, 1, SRC), lambda b: (b, 0, 0)),
        out_shape=jax.ShapeDtypeStruct((B, 1, SRC), f32),
    )(x, dp_w1r, row(dp_b1), row(dp_g1), row(dp_be1),
      dp_w2r, row(dp_b2), row(dp_g2), row(dp_be2), row(dp_lw), dp_lb.reshape(1, 1))

    gidx_flat = gidx3.reshape(_ROWS)
    table = x.reshape(B * SRC, D)

    sc_gather = functools.partial(
        pl.kernel,
        mesh=plsc.VectorSubcoreMesh(core_axis_name="c", subcore_axis_name="s",
                                    num_cores=_NC, num_subcores=_NS),
        out_type=jax.ShapeDtypeStruct((_ROWS, D), f32),
        scratch_types=[
            pltpu.VMEM((_RPW,), jnp.int32),
            pltpu.VMEM((_CH, D), f32),
            pltpu.VMEM((_CH, D), f32),
            pltpu.VMEM((_CH, D), f32),
        ] + [pltpu.SemaphoreType.DMA] * 6,
    )(_sc_gather_body)
    expanded_raw = sc_gather(table, gidx_flat).reshape(B, MAXLEN, D)

    out, en3 = pl.pallas_call(
        _stage_b_body,
        grid=(B // _BB,),
        in_specs=[
            pl.BlockSpec((_BB, MAXLEN, D), lambda b: (b, 0, 0)),
            pl.BlockSpec((_BB, 1, MAXLEN), lambda b: (b, 0, 0)),
            pl.BlockSpec((_BB, 1, MAXLEN), lambda b: (b, 0, 0)),
            _full_spec((NBINS, 1)), _full_spec((NBINS, 1)),
            _full_spec((3 * D, FILT)), _full_spec((1, FILT)),
            _full_spec((1, FILT)), _full_spec((1, FILT)),
            _full_spec((3 * FILT, FILT)), _full_spec((1, FILT)),
            _full_spec((1, FILT)), _full_spec((1, FILT)),
            _full_spec((1, FILT)), _full_spec((1, 1)),
            _full_spec((NBINS, D)),
        ],
        out_specs=[
            pl.BlockSpec((_BB, MAXLEN, D), lambda b: (b, 0, 0)),
            pl.BlockSpec((_BB, 1, MAXLEN), lambda b: (b, 0, 0)),
        ],
        out_shape=[
            jax.ShapeDtypeStruct((B, MAXLEN, D), f32),
            jax.ShapeDtypeStruct((B, 1, MAXLEN), f32),
        ],
    )(expanded_raw, mask3, e3, bounds_lo, bounds_hi,
      ep_w1r, row(ep_b1), row(ep_g1), row(ep_be1),
      ep_w2r, row(ep_b2), row(ep_g2), row(ep_be2),
      row(ep_lw), ep_lb.reshape(1, 1), emb_table)

    return (out, dp3.reshape(B, SRC), en3.reshape(B, MAXLEN))


# speedup vs baseline: 1.0140x; 1.0140x over previous
import functools

import jax
import jax.numpy as jnp
from jax import lax
from jax.experimental import pallas as pl
from jax.experimental.pallas import tpu as pltpu
from jax.experimental.pallas import tpu_sc as plsc

B, SRC, D = 16, 512, 256
FILT = 256
NBINS = 256
MAXLEN = 2048

_NC = 2
_NS = 16
_NW = _NC * _NS
_ROWS = B * MAXLEN
_RPW = _ROWS // _NW
_CH = 128
_NCHUNK = _RPW // _CH


def _conv_ln(h, w_ref, b_ref, g_ref, be_ref):
    t, d = h.shape
    f = w_ref.shape[1]
    a0 = jnp.dot(h, w_ref[0:d, :], preferred_element_type=jnp.float32,
                 precision=lax.Precision.DEFAULT)
    a1 = jnp.dot(h, w_ref[d:2 * d, :], preferred_element_type=jnp.float32,
                 precision=lax.Precision.DEFAULT)
    a2 = jnp.dot(h, w_ref[2 * d:3 * d, :], preferred_element_type=jnp.float32,
                 precision=lax.Precision.DEFAULT)
    zrow = jnp.zeros((1, f), jnp.float32)
    y = (a1 + jnp.concatenate([zrow, a0[:-1, :]], axis=0)
         + jnp.concatenate([a2[1:, :], zrow], axis=0) + b_ref[...])
    y = jnp.maximum(y, 0.0)
    m = jnp.mean(y, axis=1, keepdims=True)
    v = jnp.mean((y - m) ** 2, axis=1, keepdims=True)
    return (y - m) * lax.rsqrt(v + 1e-5) * g_ref[...] + be_ref[...]


_AB = 4
_BB = 2


def _stage_a_body(x_ref, w1_ref, b1_ref, g1_ref, be1_ref,
                  w2_ref, b2_ref, g2_ref, be2_ref, lw_ref, lb_ref,
                  dp_ref):
    for i in range(_AB):
        h = x_ref[i]
        h = _conv_ln(h, w1_ref, b1_ref, g1_ref, be1_ref)
        h = _conv_ln(h, w2_ref, b2_ref, g2_ref, be2_ref)
        dp = jnp.sum(h * lw_ref[...], axis=1, keepdims=True) + lb_ref[0, 0]
        dp_ref[i] = jnp.transpose(jnp.maximum(dp, 1e-8), (1, 0))


def _idx_body(dur_ref, gidx_ref, mask_ref):
    f32 = jnp.float32
    jj_col = lax.broadcasted_iota(jnp.int32, (SRC, 1), 0)
    ii_row = lax.broadcasted_iota(jnp.int32, (1, SRC), 1)
    umat = (jj_col <= ii_row).astype(f32)
    pos_row = lax.broadcasted_iota(jnp.int32, (1, MAXLEN), 1)
    ones_row = jnp.ones((1, SRC), f32)
    spread = jnp.bitwise_and(pos_row, SRC - 1)
    for i in range(_AB):
        b = pl.program_id(0) * _AB + i
        d_row = dur_ref[i].astype(f32)
        cum_row = jnp.dot(d_row, umat, preferred_element_type=f32,
                          precision=lax.Precision.DEFAULT)
        cum_col = jnp.transpose(cum_row, (1, 0))

        hmat = (cum_col <= pos_row.astype(f32)).astype(f32)
        idx_row = jnp.dot(ones_row, hmat, preferred_element_type=f32,
                          precision=lax.Precision.DEFAULT)
        idx_i = jnp.minimum(idx_row, SRC - 1).astype(jnp.int32)

        invalid = hmat[SRC - 1:SRC, :]
        gidx_ref[i] = jnp.where(invalid == 0.0, idx_i, spread) + b * SRC
        mask_ref[i] = 1.0 - invalid


def _stage_b_body(exp_ref, mask_ref, e_ref, bounds_lo_ref, bounds_hi_ref,
                  w1_ref, b1_ref, g1_ref, be1_ref,
                  w2_ref, b2_ref, g2_ref, be2_ref, lw_ref, lb_ref,
                  emb_t_ref, out_ref, en_ref):
    for i in range(_BB):
        m_col = jnp.transpose(mask_ref[i], (1, 0))
        e = exp_ref[i] * m_col
        h = _conv_ln(e, w1_ref, b1_ref, g1_ref, be1_ref)
        h = _conv_ln(h, w2_ref, b2_ref, g2_ref, be2_ref)
        en = jnp.sum(h * lw_ref[...], axis=1, keepdims=True) + lb_ref[0, 0]
        en_ref[i] = jnp.transpose(en, (1, 0))

        et_row = e_ref[i]
        lo = bounds_lo_ref[...] < et_row
        hi = bounds_hi_ref[...] < et_row
        onehot_t = jnp.where(lo & jnp.logical_not(hi), 1.0, 0.0)
        emb_rows = lax.dot_general(
            onehot_t, emb_t_ref[...], (((0,), (0,)), ((), ())),
            preferred_element_type=jnp.float32)
        out_ref[i] = e + emb_rows


def _sc_gather_body(table_hbm, gidx_hbm, out_hbm, idx_v,
                    rows0, rows1, rows2, gs0, gs1, gs2, ws0, ws1, ws2):
    wid = lax.axis_index("s") * _NC + lax.axis_index("c")
    base = wid * _RPW
    pltpu.sync_copy(gidx_hbm.at[pl.ds(base, _RPW)], idx_v)
    bufs = (rows0, rows1, rows2)
    gsems = (gs0, gs1, gs2)
    wsems = (ws0, ws1, ws2)
    gh, wh = {}, {}
    for c in range(_NCHUNK):
        i = c % 3
        if c >= 3:
            wh[c - 3].wait()
        gh[c] = pltpu.async_copy(
            table_hbm.at[idx_v.at[pl.ds(c * _CH, _CH)]], bufs[i], gsems[i])
        if c >= 1:
            j = (c - 1) % 3
            gh[c - 1].wait()
            wh[c - 1] = pltpu.async_copy(
                bufs[j], out_hbm.at[pl.ds(base + (c - 1) * _CH, _CH)], wsems[j])
    last = _NCHUNK - 1
    gh[last].wait()
    wh[last] = pltpu.async_copy(
        bufs[last % 3], out_hbm.at[pl.ds(base + last * _CH, _CH)], wsems[last % 3])
    for c in (last - 2, last - 1, last):
        wh[c].wait()


def _full_spec(shape):
    return pl.BlockSpec(shape, lambda b: (0,) * len(shape))


def kernel(x, max_len, duration_target, energy_target,
           dp_w1, dp_b1, dp_g1, dp_be1, dp_w2, dp_b2, dp_g2, dp_be2,
           dp_lw, dp_lb, ep_w1, ep_b1, ep_g1, ep_be1, ep_w2, ep_b2,
           ep_g2, ep_be2, ep_lw, ep_lb, emb_table, bounds):
    f32 = jnp.float32

    dur3 = duration_target.reshape(B, 1, SRC)
    e3 = energy_target.reshape(B, 1, MAXLEN)
    bounds_lo = jnp.concatenate(
        [jnp.full((1,), -jnp.inf, f32), bounds]).reshape(NBINS, 1)
    bounds_hi = jnp.concatenate(
        [bounds, jnp.full((1,), jnp.inf, f32)]).reshape(NBINS, 1)

    dp_w1r = dp_w1.reshape(3 * D, FILT)
    dp_w2r = dp_w2.reshape(3 * FILT, FILT)
    ep_w1r = ep_w1.reshape(3 * D, FILT)
    ep_w2r = ep_w2.reshape(3 * FILT, FILT)
    row = lambda a: a.reshape(1, -1)

    gidx3, mask3 = pl.pallas_call(
        _idx_body,
        grid=(B // _AB,),
        in_specs=[pl.BlockSpec((_AB, 1, SRC), lambda b: (b, 0, 0))],
        out_specs=[
            pl.BlockSpec((_AB, 1, MAXLEN), lambda b: (b, 0, 0)),
            pl.BlockSpec((_AB, 1, MAXLEN), lambda b: (b, 0, 0)),
        ],
        out_shape=[
            jax.ShapeDtypeStruct((B, 1, MAXLEN), jnp.int32),
            jax.ShapeDtypeStruct((B, 1, MAXLEN), f32),
        ],
    )(dur3)

    dp3 = pl.pallas_call(
        _stage_a_body,
        grid=(B // _AB,),
        in_specs=[
            pl.BlockSpec((_AB, SRC, D), lambda b: (b, 0, 0)),
            _full_spec((3 * D, FILT)), _full_spec((1, FILT)),
            _full_spec((1, FILT)), _full_spec((1, FILT)),
            _full_spec((3 * FILT, FILT)), _full_spec((1, FILT)),
            _full_spec((1, FILT)), _full_spec((1, FILT)),
            _full_spec((1, FILT)), _full_spec((1, 1)),
        ],
        out_specs=pl.BlockSpec((_AB, 1, SRC), lambda b: (b, 0, 0)),
        out_shape=jax.ShapeDtypeStruct((B, 1, SRC), f32),
    )(x, dp_w1r, row(dp_b1), row(dp_g1), row(dp_be1),
      dp_w2r, row(dp_b2), row(dp_g2), row(dp_be2), row(dp_lw), dp_lb.reshape(1, 1))

    gidx_flat = gidx3.reshape(_ROWS)
    table = x.reshape(B * SRC, D)

    sc_gather = functools.partial(
        pl.kernel,
        mesh=plsc.VectorSubcoreMesh(core_axis_name="c", subcore_axis_name="s",
                                    num_cores=_NC, num_subcores=_NS),
        out_type=jax.ShapeDtypeStruct((_ROWS, D), f32),
        scratch_types=[
            pltpu.VMEM((_RPW,), jnp.int32),
            pltpu.VMEM((_CH, D), f32),
            pltpu.VMEM((_CH, D), f32),
            pltpu.VMEM((_CH, D), f32),
        ] + [pltpu.SemaphoreType.DMA] * 6,
    )(_sc_gather_body)
    expanded_raw = sc_gather(table, gidx_flat).reshape(B, MAXLEN, D)

    out, en3 = pl.pallas_call(
        _stage_b_body,
        grid=(B // _BB,),
        in_specs=[
            pl.BlockSpec((_BB, MAXLEN, D), lambda b: (b, 0, 0)),
            pl.BlockSpec((_BB, 1, MAXLEN), lambda b: (b, 0, 0)),
            pl.BlockSpec((_BB, 1, MAXLEN), lambda b: (b, 0, 0)),
            _full_spec((NBINS, 1)), _full_spec((NBINS, 1)),
            _full_spec((3 * D, FILT)), _full_spec((1, FILT)),
            _full_spec((1, FILT)), _full_spec((1, FILT)),
            _full_spec((3 * FILT, FILT)), _full_spec((1, FILT)),
            _full_spec((1, FILT)), _full_spec((1, FILT)),
            _full_spec((1, FILT)), _full_spec((1, 1)),
            _full_spec((NBINS, D)),
        ],
        out_specs=[
            pl.BlockSpec((_BB, MAXLEN, D), lambda b: (b, 0, 0)),
            pl.BlockSpec((_BB, 1, MAXLEN), lambda b: (b, 0, 0)),
        ],
        out_shape=[
            jax.ShapeDtypeStruct((B, MAXLEN, D), f32),
            jax.ShapeDtypeStruct((B, 1, MAXLEN), f32),
        ],
    )(expanded_raw, mask3, e3, bounds_lo, bounds_hi,
      ep_w1r, row(ep_b1), row(ep_g1), row(ep_be1),
      ep_w2r, row(ep_b2), row(ep_g2), row(ep_be2),
      row(ep_lw), ep_lb.reshape(1, 1), emb_table)

    return (out, dp3.reshape(B, SRC), en3.reshape(B, MAXLEN))


# speedup vs baseline: 1.0162x; 1.0022x over previous
import functools

import jax
import jax.numpy as jnp
from jax import lax
from jax.experimental import pallas as pl
from jax.experimental.pallas import tpu as pltpu
from jax.experimental.pallas import tpu_sc as plsc

B, SRC, D = 16, 512, 256
FILT = 256
NBINS = 256
MAXLEN = 2048

_NC = 2
_NS = 16
_NW = _NC * _NS
_ROWS = B * MAXLEN
_RPW = _ROWS // _NW
_CH = 128
_NCHUNK = _RPW // _CH


def _conv_ln(h, w_ref, b_ref, g_ref, be_ref):
    t, d = h.shape
    f = w_ref.shape[1]
    a0 = jnp.dot(h, w_ref[0:d, :], preferred_element_type=jnp.float32,
                 precision=lax.Precision.DEFAULT)
    a1 = jnp.dot(h, w_ref[d:2 * d, :], preferred_element_type=jnp.float32,
                 precision=lax.Precision.DEFAULT)
    a2 = jnp.dot(h, w_ref[2 * d:3 * d, :], preferred_element_type=jnp.float32,
                 precision=lax.Precision.DEFAULT)
    zrow = jnp.zeros((1, f), jnp.float32)
    y = (a1 + jnp.concatenate([zrow, a0[:-1, :]], axis=0)
         + jnp.concatenate([a2[1:, :], zrow], axis=0) + b_ref[...])
    y = jnp.maximum(y, 0.0)
    m = jnp.mean(y, axis=1, keepdims=True)
    v = jnp.mean((y - m) ** 2, axis=1, keepdims=True)
    return (y - m) * lax.rsqrt(v + 1e-5) * g_ref[...] + be_ref[...]


_AB = 4
_BB = 2


def _stage_a_body(x_ref, w1_ref, b1_ref, g1_ref, be1_ref,
                  w2_ref, b2_ref, g2_ref, be2_ref, lw_ref, lb_ref,
                  dp_ref):
    for i in range(_AB):
        h = x_ref[i]
        h = _conv_ln(h, w1_ref, b1_ref, g1_ref, be1_ref)
        h = _conv_ln(h, w2_ref, b2_ref, g2_ref, be2_ref)
        dp = jnp.sum(h * lw_ref[...], axis=1, keepdims=True) + lb_ref[0, 0]
        dp_ref[i] = jnp.transpose(jnp.maximum(dp, 1e-8), (1, 0))


def _idx_body(dur_ref, gidx_ref, mask_ref):
    f32 = jnp.float32
    jj_col = lax.broadcasted_iota(jnp.int32, (SRC, 1), 0)
    ii_row = lax.broadcasted_iota(jnp.int32, (1, SRC), 1)
    umat = (jj_col <= ii_row).astype(f32)
    pos_row = lax.broadcasted_iota(jnp.int32, (1, MAXLEN), 1)
    ones_row = jnp.ones((1, SRC), f32)
    spread = jnp.bitwise_and(pos_row, SRC - 1)
    for i in range(_AB):
        b = pl.program_id(0) * _AB + i
        d_row = dur_ref[i].astype(f32)
        cum_row = jnp.dot(d_row, umat, preferred_element_type=f32,
                          precision=lax.Precision.DEFAULT)
        cum_col = jnp.transpose(cum_row, (1, 0))

        hmat = (cum_col <= pos_row.astype(f32)).astype(f32)
        idx_row = jnp.dot(ones_row, hmat, preferred_element_type=f32,
                          precision=lax.Precision.DEFAULT)
        idx_i = jnp.minimum(idx_row, SRC - 1).astype(jnp.int32)

        invalid = hmat[SRC - 1:SRC, :]
        gidx_ref[i] = jnp.where(invalid == 0.0, idx_i, spread) + b * SRC
        mask_ref[i] = 1.0 - invalid


def _stage_b_body(exp_ref, mask_ref, e_ref, bounds_lo_ref, bounds_hi_ref,
                  w1_ref, b1_ref, g1_ref, be1_ref,
                  w2_ref, b2_ref, g2_ref, be2_ref, lw_ref, lb_ref,
                  emb_ref, out_ref, en_ref):
    for i in range(_BB):
        m_col = jnp.transpose(mask_ref[i], (1, 0))
        e = exp_ref[i] * m_col
        h = _conv_ln(e, w1_ref, b1_ref, g1_ref, be1_ref)
        h = _conv_ln(h, w2_ref, b2_ref, g2_ref, be2_ref)
        en = jnp.sum(h * lw_ref[...], axis=1, keepdims=True) + lb_ref[0, 0]
        en_ref[i] = jnp.transpose(en, (1, 0))

        et_row = e_ref[i]
        lo = bounds_lo_ref[...] < et_row
        hi = bounds_hi_ref[...] < et_row
        onehot_t = jnp.where(lo & jnp.logical_not(hi), 1.0, 0.0)
        emb_rows = lax.dot_general(
            onehot_t, emb_ref[...], (((0,), (0,)), ((), ())),
            preferred_element_type=jnp.float32)
        out_ref[i] = e + emb_rows


def _sc_gather_body(table_hbm, gidx_hbm, out_hbm, idx_v,
                    rows0, rows1, rows2, gs0, gs1, gs2, ws0, ws1, ws2):
    wid = lax.axis_index("s") * _NC + lax.axis_index("c")
    base = wid * _RPW
    pltpu.sync_copy(gidx_hbm.at[pl.ds(base, _RPW)], idx_v)
    bufs = (rows0, rows1, rows2)
    gsems = (gs0, gs1, gs2)
    wsems = (ws0, ws1, ws2)
    gh, wh = {}, {}
    for c in range(_NCHUNK):
        i = c % 3
        if c >= 3:
            wh[c - 3].wait()
        gh[c] = pltpu.async_copy(
            table_hbm.at[idx_v.at[pl.ds(c * _CH, _CH)]], bufs[i], gsems[i])
        if c >= 1:
            j = (c - 1) % 3
            gh[c - 1].wait()
            wh[c - 1] = pltpu.async_copy(
                bufs[j], out_hbm.at[pl.ds(base + (c - 1) * _CH, _CH)], wsems[j])
    last = _NCHUNK - 1
    gh[last].wait()
    wh[last] = pltpu.async_copy(
        bufs[last % 3], out_hbm.at[pl.ds(base + last * _CH, _CH)], wsems[last % 3])
    for c in (last - 2, last - 1, last):
        wh[c].wait()


def _full_spec(shape):
    return pl.BlockSpec(shape, lambda b: (0,) * len(shape))


def kernel(x, max_len, duration_target, energy_target,
           dp_w1, dp_b1, dp_g1, dp_be1, dp_w2, dp_b2, dp_g2, dp_be2,
           dp_lw, dp_lb, ep_w1, ep_b1, ep_g1, ep_be1, ep_w2, ep_b2,
           ep_g2, ep_be2, ep_lw, ep_lb, emb_table, bounds):
    f32 = jnp.float32

    dur3 = duration_target.reshape(B, 1, SRC)
    e3 = energy_target.reshape(B, 1, MAXLEN)
    bounds_lo = jnp.concatenate(
        [jnp.full((1,), -jnp.inf, f32), bounds]).reshape(NBINS, 1)
    bounds_hi = jnp.concatenate(
        [bounds, jnp.full((1,), jnp.inf, f32)]).reshape(NBINS, 1)

    dp_w1r = dp_w1.reshape(3 * D, FILT)
    dp_w2r = dp_w2.reshape(3 * FILT, FILT)
    ep_w1r = ep_w1.reshape(3 * D, FILT)
    ep_w2r = ep_w2.reshape(3 * FILT, FILT)
    row = lambda a: a.reshape(1, -1)

    gidx3, mask3 = pl.pallas_call(
        _idx_body,
        grid=(B // _AB,),
        in_specs=[pl.BlockSpec((_AB, 1, SRC), lambda b: (b, 0, 0))],
        out_specs=[
            pl.BlockSpec((_AB, 1, MAXLEN), lambda b: (b, 0, 0)),
            pl.BlockSpec((_AB, 1, MAXLEN), lambda b: (b, 0, 0)),
        ],
        out_shape=[
            jax.ShapeDtypeStruct((B, 1, MAXLEN), jnp.int32),
            jax.ShapeDtypeStruct((B, 1, MAXLEN), f32),
        ],
    )(dur3)

    dp3 = pl.pallas_call(
        _stage_a_body,
        grid=(B // _AB,),
        in_specs=[
            pl.BlockSpec((_AB, SRC, D), lambda b: (b, 0, 0)),
            _full_spec((3 * D, FILT)), _full_spec((1, FILT)),
            _full_spec((1, FILT)), _full_spec((1, FILT)),
            _full_spec((3 * FILT, FILT)), _full_spec((1, FILT)),
            _full_spec((1, FILT)), _full_spec((1, FILT)),
            _full_spec((1, FILT)), _full_spec((1, 1)),
        ],
        out_specs=pl.BlockSpec((_AB, 1, SRC), lambda b: (b, 0, 0)),
        out_shape=jax.ShapeDtypeStruct((B, 1, SRC), f32),
    )(x, dp_w1r, row(dp_b1), row(dp_g1), row(dp_be1),
      dp_w2r, row(dp_b2), row(dp_g2), row(dp_be2), row(dp_lw), dp_lb.reshape(1, 1))

    gidx_flat = gidx3.reshape(_ROWS)
    table = x.reshape(B * SRC, D)

    sc_gather = functools.partial(
        pl.kernel,
        mesh=plsc.VectorSubcoreMesh(core_axis_name="c", subcore_axis_name="s",
                                    num_cores=_NC, num_subcores=_NS),
        out_type=jax.ShapeDtypeStruct((_ROWS, D), f32),
        scratch_types=[
            pltpu.VMEM((_RPW,), jnp.int32),
            pltpu.VMEM((_CH, D), f32),
            pltpu.VMEM((_CH, D), f32),
            pltpu.VMEM((_CH, D), f32),
        ] + [pltpu.SemaphoreType.DMA] * 6,
    )(_sc_gather_body)
    expanded_raw = sc_gather(table, gidx_flat).reshape(B, MAXLEN, D)

    out, en3 = pl.pallas_call(
        _stage_b_body,
        grid=(B // _BB,),
        in_specs=[
            pl.BlockSpec((_BB, MAXLEN, D), lambda b: (b, 0, 0)),
            pl.BlockSpec((_BB, 1, MAXLEN), lambda b: (b, 0, 0)),
            pl.BlockSpec((_BB, 1, MAXLEN), lambda b: (b, 0, 0)),
            _full_spec((NBINS, 1)), _full_spec((NBINS, 1)),
            _full_spec((3 * D, FILT)), _full_spec((1, FILT)),
            _full_spec((1, FILT)), _full_spec((1, FILT)),
            _full_spec((3 * FILT, FILT)), _full_spec((1, FILT)),
            _full_spec((1, FILT)), _full_spec((1, FILT)),
            _full_spec((1, FILT)), _full_spec((1, 1)),
            _full_spec((NBINS, D)),
        ],
        out_specs=[
            pl.BlockSpec((_BB, MAXLEN, D), lambda b: (b, 0, 0)),
            pl.BlockSpec((_BB, 1, MAXLEN), lambda b: (b, 0, 0)),
        ],
        out_shape=[
            jax.ShapeDtypeStruct((B, MAXLEN, D), f32),
            jax.ShapeDtypeStruct((B, 1, MAXLEN), f32),
        ],
    )(expanded_raw, mask3, e3, bounds_lo, bounds_hi,
      ep_w1r, row(ep_b1), row(ep_g1), row(ep_be1),
      ep_w2r, row(ep_b2), row(ep_g2), row(ep_be2),
      row(ep_lw), ep_lb.reshape(1, 1), emb_table)

    return (out, dp3.reshape(B, SRC), en3.reshape(B, MAXLEN))
